# Initial kernel scaffold; baseline (speedup 1.0000x reference)
import jax
import jax.numpy as jnp
from jax.experimental import pallas as pl


def kernel(atomic_numbers, pos, edge_index, edge_attr, emb, e_w1, e_b1, e_w2, e_b2, h_w1, h_b1, h_w2, h_b2, x_w1, x_b1, x_w2, x_b2):
    raise NotImplementedError("write your pallas kernel here")


# speedup vs baseline: 2.4721x; 2.4721x over previous
import functools

import jax
import jax.numpy as jnp
from jax import lax
from jax.experimental import pallas as pl
from jax.experimental.pallas import tpu as pltpu
from jax.experimental.pallas import tpu_sc as plsc

N = 10000
E = 320000
H = 128
NC = 2
NS = 16
NW = NC * NS
EPT = E // NW
CH = 80
NCH = EPT // CH
NPT = N // NS
NB = 400
NGRID = N // NB
EB = 512
EGRID = E // EB
PW = 16

def _sc_mesh():
    return plsc.VectorSubcoreMesh(core_axis_name="c", subcore_axis_name="s",
                                  num_cores=NC, num_subcores=NS)


def _emb_body(ids_ref, emb_ref, out_ref):
    ids = ids_ref[...]
    cols = lax.broadcasted_iota(jnp.int32, (NB, H), 1)
    onehot = (cols == ids).astype(jnp.float32)
    out_ref[...] = jnp.dot(onehot, emb_ref[...],
                           preferred_element_type=jnp.float32)


def _embed(ids2d, embp):
    return pl.pallas_call(
        _emb_body,
        grid=(NGRID,),
        in_specs=[pl.BlockSpec((NB, 1), lambda i: (i, 0)),
                  pl.BlockSpec((H, H), lambda i: (0, 0))],
        out_specs=pl.BlockSpec((NB, H), lambda i: (i, 0)),
        out_shape=jax.ShapeDtypeStruct((N, H), jnp.float32),
    )(ids2d, embp)


def _gather_body(h_hbm, xf_hbm, src_hbm, dst_hbm,
                 hi_hbm, hj_hbm, g_hbm,
                 xtab, sidx, didx, hibuf, hjbuf, gbuf, sem):
    c = lax.axis_index("c")
    s = lax.axis_index("s")
    base = (c * NS + s) * EPT
    pltpu.sync_copy(xf_hbm, xtab)
    iota16 = lax.iota(jnp.int32, 16)

    def chunk(j, carry):
        off = base + j * CH
        pltpu.sync_copy(src_hbm.at[pl.ds(off, CH)], sidx)
        pltpu.sync_copy(dst_hbm.at[pl.ds(off, CH)], didx)
        cp1 = pltpu.async_copy(h_hbm.at[sidx], hjbuf, sem)
        cp2 = pltpu.async_copy(h_hbm.at[didx], hibuf, sem)
        for q in range(CH // 16):
            s16 = sidx[pl.ds(q * 16, 16)]
            d16 = didx[pl.ds(q * 16, 16)]
            comps = []
            for cc in range(3):
                xs = plsc.load_gather(xtab, [s16 * 4 + cc])
                xd = plsc.load_gather(xtab, [d16 * 4 + cc])
                comps.append(xs - xd)
            r2 = comps[0] * comps[0] + comps[1] * comps[1] + comps[2] * comps[2]
            gb = (q * 16 + iota16) * 8
            for cc in range(3):
                plsc.store_scatter(gbuf, [gb + cc], comps[cc])
            plsc.store_scatter(gbuf, [gb + 3], r2)
        cp1.wait()
        cp2.wait()
        pltpu.sync_copy(hibuf, hi_hbm.at[pl.ds(off, CH)])
        pltpu.sync_copy(hjbuf, hj_hbm.at[pl.ds(off, CH)])
        pltpu.sync_copy(gbuf, g_hbm.at[pl.ds(off * 8, CH * 8)])
        return carry

    lax.fori_loop(0, NCH, chunk, 0)


def _sc_gather(h, xflat, src, dst):
    f = pl.kernel(
        _gather_body,
        out_type=(jax.ShapeDtypeStruct((E, H), jnp.float32),
                  jax.ShapeDtypeStruct((E, H), jnp.float32),
                  jax.ShapeDtypeStruct((E * 8,), jnp.float32)),
        mesh=_sc_mesh(),
        compiler_params=pltpu.CompilerParams(needs_layout_passes=False),
        scratch_types=[pltpu.VMEM((4 * N,), jnp.float32),
                       pltpu.VMEM((CH,), jnp.int32),
                       pltpu.VMEM((CH,), jnp.int32),
                       pltpu.VMEM((CH, H), jnp.float32),
                       pltpu.VMEM((CH, H), jnp.float32),
                       pltpu.VMEM((CH * 8,), jnp.float32),
                       pltpu.SemaphoreType.DMA],
    )
    return f(h, xflat, src, dst)


def _edge_body(hi_ref, hj_ref, g_ref, A_ref, B_ref, c_ref, b1_ref,
               W2_ref, b2_ref, xw1_ref, xb1_ref, xw2_ref, xb2_ref,
               mij_ref, p_ref):
    g = g_ref[...]
    r = jnp.sqrt(g[:, 3:4])
    t = (jnp.dot(hi_ref[...], A_ref[...], preferred_element_type=jnp.float32)
         + jnp.dot(hj_ref[...], B_ref[...], preferred_element_type=jnp.float32)
         + r * c_ref[...] + b1_ref[...])
    m = t * jax.nn.sigmoid(t)
    t2 = jnp.dot(m, W2_ref[...], preferred_element_type=jnp.float32) + b2_ref[...]
    m2 = t2 * jax.nn.sigmoid(t2)
    mij_ref[...] = m2
    t3 = jnp.dot(m2, xw1_ref[...], preferred_element_type=jnp.float32) + xb1_ref[...]
    w = t3 * jax.nn.sigmoid(t3)
    Wij = jnp.dot(w, xw2_ref[...], preferred_element_type=jnp.float32) + xb2_ref[...]
    p = Wij * g[:, 0:3]
    p_ref[...] = jnp.concatenate(
        [p, jnp.zeros((EB, PW - 3), jnp.float32)], axis=1)


def _edge_mlp(hi, hj, g, A, B, crow, b1, W2, b2, xw1, xb1, xw2, xb2):
    full = lambda shape: pl.BlockSpec(shape, lambda i: tuple(0 for _ in shape))
    return pl.pallas_call(
        _edge_body,
        grid=(EGRID,),
        in_specs=[pl.BlockSpec((EB, H), lambda i: (i, 0)),
                  pl.BlockSpec((EB, H), lambda i: (i, 0)),
                  pl.BlockSpec((EB, 8), lambda i: (i, 0)),
                  full((H, H)), full((H, H)), full((1, H)), full((1, H)),
                  full((H, H)), full((1, H)),
                  full((H, H)), full((1, H)), full((H, 1)), full((1, 1))],
        out_specs=[pl.BlockSpec((EB, H), lambda i: (i, 0)),
                   pl.BlockSpec((EB, PW), lambda i: (i, 0))],
        out_shape=[jax.ShapeDtypeStruct((E, H), jnp.float32),
                   jax.ShapeDtypeStruct((E, PW), jnp.float32)],
    )(hi, hj, g, A, B, crow, b1, W2, b2, xw1, xb1, xw2, xb2)


def _scatter_m_body(mij_hbm, dst_hbm, zh_hbm, mpart_hbm, didx, mbuf, macc):
    c = lax.axis_index("c")
    s = lax.axis_index("s")
    base = (c * NS + s) * EPT
    rows0 = s * 624

    @pl.when(s < NS - 1)
    def _():
        pltpu.sync_copy(zh_hbm.at[pl.ds(rows0, 624)], macc.at[pl.ds(rows0, 624)])

    @pl.when(s == NS - 1)
    def _():
        pltpu.sync_copy(zh_hbm.at[pl.ds(rows0, 640)], macc.at[pl.ds(rows0, 640)])

    plsc.subcore_barrier()

    def chunk(j, carry):
        off = base + j * CH
        pltpu.sync_copy(dst_hbm.at[pl.ds(off, CH)], didx)
        pltpu.sync_copy(mij_hbm.at[pl.ds(off, CH)], mbuf)
        pltpu.sync_copy(mbuf, macc.at[didx], add=True)
        return carry

    lax.fori_loop(0, NCH, chunk, 0)
    plsc.subcore_barrier()

    @pl.when(s < NS - 1)
    def _():
        pltpu.sync_copy(macc.at[pl.ds(rows0, 624)],
                        mpart_hbm.at[c].at[pl.ds(rows0, 624)])

    @pl.when(s == NS - 1)
    def _():
        pltpu.sync_copy(macc.at[pl.ds(rows0, 640)],
                        mpart_hbm.at[c].at[pl.ds(rows0, 640)])


def _sc_scatter_m(mij, dst, zh):
    f = pl.kernel(
        _scatter_m_body,
        out_type=jax.ShapeDtypeStruct((NC, N, H), jnp.float32),
        mesh=_sc_mesh(),
        compiler_params=pltpu.CompilerParams(needs_layout_passes=False),
        scratch_types=[pltpu.VMEM((CH,), jnp.int32),
                       pltpu.VMEM((CH, H), jnp.float32),
                       pltpu.VMEM_SHARED((N, H), jnp.float32)],
    )
    return f(mij, dst, zh)


def _scatter_x_body(pf_hbm, dst_hbm, xpart_hbm, didx, pbuf, xacc):
    c = lax.axis_index("c")
    s = lax.axis_index("s")
    base = (c * NS + s) * EPT
    iota16 = lax.iota(jnp.int32, 16)

    def zloop(i, carry):
        plsc.store_scatter(xacc, [i * 16 + iota16], jnp.zeros((16,), jnp.float32))
        return carry

    lax.fori_loop(0, (N * 4) // 16, zloop, 0)

    def chunk(j, carry):
        off = base + j * CH
        pltpu.sync_copy(dst_hbm.at[pl.ds(off, CH)], didx)
        pltpu.sync_copy(pf_hbm.at[pl.ds(off * PW, CH * PW)], pbuf)
        for q in range(CH // 16):
            d16 = didx[pl.ds(q * 16, 16)]
            for cc in range(3):
                val = plsc.load_gather(pbuf, [(q * 16 + iota16) * PW + cc])
                plsc.addupdate_scatter(xacc, [d16 * 4 + cc], val)
        return carry

    lax.fori_loop(0, NCH, chunk, 0)
    wid = c * NS + s
    pltpu.sync_copy(xacc, xpart_hbm.at[wid])


def _sc_scatter_x(pflat, dst):
    f = pl.kernel(
        _scatter_x_body,
        out_type=jax.ShapeDtypeStruct((NW, N * 4), jnp.float32),
        mesh=_sc_mesh(),
        compiler_params=pltpu.CompilerParams(needs_layout_passes=False),
        scratch_types=[pltpu.VMEM((CH,), jnp.int32),
                       pltpu.VMEM((CH * PW,), jnp.float32),
                       pltpu.VMEM((N * 4,), jnp.float32)],
    )
    return f(pflat, dst)


def _xsum_body(x_ref, xp_ref, out_ref):
    out_ref[...] = x_ref[...] + jnp.sum(xp_ref[...], axis=0)


def _xsum(x4, xpart):
    XL = 1600
    x3 = x4.reshape(NGRID, 1, XL)
    xp4 = xpart.reshape(NW, NGRID, 1, XL)
    out = pl.pallas_call(
        _xsum_body,
        grid=(NGRID,),
        in_specs=[pl.BlockSpec((1, 1, XL), lambda i: (i, 0, 0)),
                  pl.BlockSpec((NW, 1, 1, XL), lambda i: (0, i, 0, 0))],
        out_specs=pl.BlockSpec((1, 1, XL), lambda i: (i, 0, 0)),
        out_shape=jax.ShapeDtypeStruct((NGRID, 1, XL), jnp.float32),
    )(x3, xp4)
    return out.reshape(N, 4)


def _node_body(h_ref, mp_ref, U_ref, V_ref, b1_ref,
               W2_ref, b2_ref, hout_ref):
    m_i = mp_ref[0] + mp_ref[1]
    t = (jnp.dot(h_ref[...], U_ref[...], preferred_element_type=jnp.float32)
         + jnp.dot(m_i, V_ref[...], preferred_element_type=jnp.float32)
         + b1_ref[...])
    hh = t * jax.nn.sigmoid(t)
    hout_ref[...] = (h_ref[...]
                     + jnp.dot(hh, W2_ref[...], preferred_element_type=jnp.float32)
                     + b2_ref[...])


def _node_update(h, mpart, U, V, hb1, hW2, hb2):
    full = lambda shape: pl.BlockSpec(shape, lambda i: tuple(0 for _ in shape))
    return pl.pallas_call(
        _node_body,
        grid=(NGRID,),
        in_specs=[pl.BlockSpec((NB, H), lambda i: (i, 0)),
                  pl.BlockSpec((NC, NB, H), lambda i: (0, i, 0)),
                  full((H, H)), full((H, H)), full((1, H)),
                  full((H, H)), full((1, H))],
        out_specs=pl.BlockSpec((NB, H), lambda i: (i, 0)),
        out_shape=jax.ShapeDtypeStruct((N, H), jnp.float32),
    )(h, mpart, U, V, hb1, hW2, hb2)


def kernel(atomic_numbers, pos, edge_index, edge_attr, emb,
           e_w1, e_b1, e_w2, e_b2,
           h_w1, h_b1, h_w2, h_b2,
           x_w1, x_b1, x_w2, x_b2):
    del edge_attr
    ids2d = atomic_numbers.astype(jnp.int32).reshape(N, 1)
    embp = jnp.zeros((H, H), jnp.float32).at[:emb.shape[0]].set(emb)
    src = edge_index[0].astype(jnp.int32)
    dst = edge_index[1].astype(jnp.int32)
    zh = jnp.zeros((N, H), jnp.float32)

    h = _embed(ids2d, embp)
    x4 = jnp.pad(pos, ((0, 0), (0, 1)))

    for l in range(e_w1.shape[0]):
        A = e_w1[l, :H]
        B = e_w1[l, H:2 * H]
        crow = e_w1[l, 2 * H:2 * H + 1]
        b1 = e_b1[l].reshape(1, H)
        W2 = e_w2[l]
        b2 = e_b2[l].reshape(1, H)
        xw1 = x_w1[l]
        xb1 = x_b1[l].reshape(1, H)
        xw2 = x_w2[l]
        xb2 = x_b2[l].reshape(1, 1)
        U = h_w1[l, :H]
        V = h_w1[l, H:]
        hb1 = h_b1[l].reshape(1, H)
        hW2 = h_w2[l]
        hb2 = h_b2[l].reshape(1, H)

        hi, hj, gflat = _sc_gather(h, x4.reshape(-1), src, dst)
        g = gflat.reshape(E, 8)
        mij, parr = _edge_mlp(hi, hj, g, A, B, crow, b1, W2, b2,
                              xw1, xb1, xw2, xb2)
        mpart = _sc_scatter_m(mij, dst, zh)
        xpart = _sc_scatter_x(parr.reshape(-1), dst)
        x4 = _xsum(x4, xpart)
        h = _node_update(h, mpart, U, V, hb1, hW2, hb2)

    return (h, x4[:, :3])


# speedup vs baseline: 3.2850x; 1.3288x over previous
import functools

import jax
import jax.numpy as jnp
from jax import lax
from jax.experimental import pallas as pl
from jax.experimental.pallas import tpu as pltpu
from jax.experimental.pallas import tpu_sc as plsc

N = 10000
E = 320000
H = 128
NC = 2
NS = 16
NW = NC * NS
EPT = E // NW
CH = 80
NCH = EPT // CH
NPT = N // NS
NB = 400
NGRID = N // NB
EB = 2560
EGRID = E // EB
PW = 16

def _sc_mesh():
    return plsc.VectorSubcoreMesh(core_axis_name="c", subcore_axis_name="s",
                                  num_cores=NC, num_subcores=NS)


def _emb_body(ids_ref, emb_ref, out_ref):
    ids = ids_ref[...]
    cols = lax.broadcasted_iota(jnp.int32, (NB, H), 1)
    onehot = (cols == ids).astype(jnp.float32)
    out_ref[...] = jnp.dot(onehot, emb_ref[...],
                           preferred_element_type=jnp.float32)


def _embed(ids2d, embp):
    return pl.pallas_call(
        _emb_body,
        grid=(NGRID,),
        in_specs=[pl.BlockSpec((NB, 1), lambda i: (i, 0)),
                  pl.BlockSpec((H, H), lambda i: (0, 0))],
        out_specs=pl.BlockSpec((NB, H), lambda i: (i, 0)),
        out_shape=jax.ShapeDtypeStruct((N, H), jnp.float32),
    )(ids2d, embp)


def _gather_body(h_hbm, xf_hbm, src_hbm, dst_hbm,
                 hi_hbm, hj_hbm, g_hbm,
                 xtab, sidx, didx, hibuf, hjbuf, gbuf, sem):
    c = lax.axis_index("c")
    s = lax.axis_index("s")
    base = (c * NS + s) * EPT
    pltpu.sync_copy(xf_hbm, xtab)
    iota16 = lax.iota(jnp.int32, 16)

    def chunk(j, carry):
        off = base + j * CH
        pltpu.sync_copy(src_hbm.at[pl.ds(off, CH)], sidx)
        pltpu.sync_copy(dst_hbm.at[pl.ds(off, CH)], didx)
        cp1 = pltpu.async_copy(h_hbm.at[sidx], hjbuf, sem)
        cp2 = pltpu.async_copy(h_hbm.at[didx], hibuf, sem)
        for q in range(CH // 16):
            s16 = sidx[pl.ds(q * 16, 16)]
            d16 = didx[pl.ds(q * 16, 16)]
            comps = []
            for cc in range(3):
                xs = plsc.load_gather(xtab, [s16 * 4 + cc])
                xd = plsc.load_gather(xtab, [d16 * 4 + cc])
                comps.append(xs - xd)
            r2 = comps[0] * comps[0] + comps[1] * comps[1] + comps[2] * comps[2]
            r16 = q * 16 + iota16
            for cc in range(3):
                plsc.store_scatter(gbuf, [r16, iota16 * 0 + cc], comps[cc])
            plsc.store_scatter(gbuf, [r16, iota16 * 0 + 3], r2)
        cp1.wait()
        cp2.wait()
        pltpu.sync_copy(hibuf, hi_hbm.at[pl.ds(off, CH)])
        pltpu.sync_copy(hjbuf, hj_hbm.at[pl.ds(off, CH)])
        pltpu.sync_copy(gbuf, g_hbm.at[pl.ds(off, CH)])
        return carry

    lax.fori_loop(0, NCH, chunk, 0)


def _sc_gather(h, xflat, src, dst):
    f = pl.kernel(
        _gather_body,
        out_type=(jax.ShapeDtypeStruct((E, H), jnp.float32),
                  jax.ShapeDtypeStruct((E, H), jnp.float32),
                  jax.ShapeDtypeStruct((E, 8), jnp.float32)),
        mesh=_sc_mesh(),
        compiler_params=pltpu.CompilerParams(needs_layout_passes=False),
        scratch_types=[pltpu.VMEM((4 * N,), jnp.float32),
                       pltpu.VMEM((CH,), jnp.int32),
                       pltpu.VMEM((CH,), jnp.int32),
                       pltpu.VMEM((CH, H), jnp.float32),
                       pltpu.VMEM((CH, H), jnp.float32),
                       pltpu.VMEM((CH, 8), jnp.float32),
                       pltpu.SemaphoreType.DMA],
    )
    return f(h, xflat, src, dst)


def _edge_body(hi_ref, hj_ref, g_ref, A_ref, B_ref, c_ref, b1_ref,
               W2_ref, b2_ref, xw1_ref, xb1_ref, xw2_ref, xb2_ref,
               mij_ref, p_ref):
    g = g_ref[...]
    r = jnp.sqrt(g[:, 3:4])
    t = (jnp.dot(hi_ref[...], A_ref[...], preferred_element_type=jnp.float32)
         + jnp.dot(hj_ref[...], B_ref[...], preferred_element_type=jnp.float32)
         + r * c_ref[...] + b1_ref[...])
    m = t * jax.nn.sigmoid(t)
    t2 = jnp.dot(m, W2_ref[...], preferred_element_type=jnp.float32) + b2_ref[...]
    m2 = t2 * jax.nn.sigmoid(t2)
    mij_ref[...] = m2
    t3 = jnp.dot(m2, xw1_ref[...], preferred_element_type=jnp.float32) + xb1_ref[...]
    w = t3 * jax.nn.sigmoid(t3)
    Wij = jnp.dot(w, xw2_ref[...], preferred_element_type=jnp.float32) + xb2_ref[...]
    p = Wij * g[:, 0:3]
    p_ref[...] = jnp.concatenate(
        [p, jnp.zeros((EB, PW - 3), jnp.float32)], axis=1)


def _edge_mlp(hi, hj, g, A, B, crow, b1, W2, b2, xw1, xb1, xw2, xb2):
    full = lambda shape: pl.BlockSpec(shape, lambda i: tuple(0 for _ in shape))
    return pl.pallas_call(
        _edge_body,
        grid=(EGRID,),
        in_specs=[pl.BlockSpec((EB, H), lambda i: (i, 0)),
                  pl.BlockSpec((EB, H), lambda i: (i, 0)),
                  pl.BlockSpec((EB, 8), lambda i: (i, 0)),
                  full((H, H)), full((H, H)), full((1, H)), full((1, H)),
                  full((H, H)), full((1, H)),
                  full((H, H)), full((1, H)), full((H, 1)), full((1, 1))],
        out_specs=[pl.BlockSpec((EB, H), lambda i: (i, 0)),
                   pl.BlockSpec((EB, PW), lambda i: (i, 0))],
        out_shape=[jax.ShapeDtypeStruct((E, H), jnp.float32),
                   jax.ShapeDtypeStruct((E, PW), jnp.float32)],
    )(hi, hj, g, A, B, crow, b1, W2, b2, xw1, xb1, xw2, xb2)


def _scatter_m_body(mij_hbm, dst_hbm, zh_hbm, mpart_hbm, didx, mbuf, macc):
    c = lax.axis_index("c")
    s = lax.axis_index("s")
    base = (c * NS + s) * EPT
    rows0 = s * 624

    @pl.when(s < NS - 1)
    def _():
        pltpu.sync_copy(zh_hbm.at[pl.ds(rows0, 624)], macc.at[pl.ds(rows0, 624)])

    @pl.when(s == NS - 1)
    def _():
        pltpu.sync_copy(zh_hbm.at[pl.ds(rows0, 640)], macc.at[pl.ds(rows0, 640)])

    plsc.subcore_barrier()

    def chunk(j, carry):
        off = base + j * CH
        pltpu.sync_copy(dst_hbm.at[pl.ds(off, CH)], didx)
        pltpu.sync_copy(mij_hbm.at[pl.ds(off, CH)], mbuf)
        pltpu.sync_copy(mbuf, macc.at[didx], add=True)
        return carry

    lax.fori_loop(0, NCH, chunk, 0)
    plsc.subcore_barrier()

    @pl.when(s < NS - 1)
    def _():
        pltpu.sync_copy(macc.at[pl.ds(rows0, 624)],
                        mpart_hbm.at[c].at[pl.ds(rows0, 624)])

    @pl.when(s == NS - 1)
    def _():
        pltpu.sync_copy(macc.at[pl.ds(rows0, 640)],
                        mpart_hbm.at[c].at[pl.ds(rows0, 640)])


def _sc_scatter_m(mij, dst, zh):
    f = pl.kernel(
        _scatter_m_body,
        out_type=jax.ShapeDtypeStruct((NC, N, H), jnp.float32),
        mesh=_sc_mesh(),
        compiler_params=pltpu.CompilerParams(needs_layout_passes=False),
        scratch_types=[pltpu.VMEM((CH,), jnp.int32),
                       pltpu.VMEM((CH, H), jnp.float32),
                       pltpu.VMEM_SHARED((N, H), jnp.float32)],
    )
    return f(mij, dst, zh)


def _scatter_x_body(pf_hbm, dst_hbm, xpart_hbm, didx, pbuf, xacc):
    c = lax.axis_index("c")
    s = lax.axis_index("s")
    base = (c * NS + s) * EPT
    iota16 = lax.iota(jnp.int32, 16)

    def zloop(i, carry):
        plsc.store_scatter(xacc, [i * 16 + iota16], jnp.zeros((16,), jnp.float32))
        return carry

    lax.fori_loop(0, (N * 4) // 16, zloop, 0)

    def chunk(j, carry):
        off = base + j * CH
        pltpu.sync_copy(dst_hbm.at[pl.ds(off, CH)], didx)
        pltpu.sync_copy(pf_hbm.at[pl.ds(off, CH)], pbuf)
        for q in range(CH // 16):
            d16 = didx[pl.ds(q * 16, 16)]
            r16 = q * 16 + iota16
            for cc in range(3):
                val = plsc.load_gather(pbuf, [r16, iota16 * 0 + cc])
                plsc.addupdate_scatter(xacc, [d16 * 4 + cc], val)
        return carry

    lax.fori_loop(0, NCH, chunk, 0)
    wid = c * NS + s
    pltpu.sync_copy(xacc, xpart_hbm.at[wid])


def _sc_scatter_x(parr, dst):
    f = pl.kernel(
        _scatter_x_body,
        out_type=jax.ShapeDtypeStruct((NW, N * 4), jnp.float32),
        mesh=_sc_mesh(),
        compiler_params=pltpu.CompilerParams(needs_layout_passes=False),
        scratch_types=[pltpu.VMEM((CH,), jnp.int32),
                       pltpu.VMEM((CH, PW), jnp.float32),
                       pltpu.VMEM((N * 4,), jnp.float32)],
    )
    return f(parr, dst)


def _xsum_body(x_ref, xp_ref, out_ref):
    out_ref[...] = x_ref[...] + jnp.sum(xp_ref[...], axis=0)


def _xsum(x4, xpart):
    XL = 1600
    x3 = x4.reshape(NGRID, 1, XL)
    xp4 = xpart.reshape(NW, NGRID, 1, XL)
    out = pl.pallas_call(
        _xsum_body,
        grid=(NGRID,),
        in_specs=[pl.BlockSpec((1, 1, XL), lambda i: (i, 0, 0)),
                  pl.BlockSpec((NW, 1, 1, XL), lambda i: (0, i, 0, 0))],
        out_specs=pl.BlockSpec((1, 1, XL), lambda i: (i, 0, 0)),
        out_shape=jax.ShapeDtypeStruct((NGRID, 1, XL), jnp.float32),
    )(x3, xp4)
    return out.reshape(N, 4)


def _node_body(h_ref, mp_ref, U_ref, V_ref, b1_ref,
               W2_ref, b2_ref, hout_ref):
    m_i = mp_ref[0] + mp_ref[1]
    t = (jnp.dot(h_ref[...], U_ref[...], preferred_element_type=jnp.float32)
         + jnp.dot(m_i, V_ref[...], preferred_element_type=jnp.float32)
         + b1_ref[...])
    hh = t * jax.nn.sigmoid(t)
    hout_ref[...] = (h_ref[...]
                     + jnp.dot(hh, W2_ref[...], preferred_element_type=jnp.float32)
                     + b2_ref[...])


def _node_update(h, mpart, U, V, hb1, hW2, hb2):
    full = lambda shape: pl.BlockSpec(shape, lambda i: tuple(0 for _ in shape))
    return pl.pallas_call(
        _node_body,
        grid=(NGRID,),
        in_specs=[pl.BlockSpec((NB, H), lambda i: (i, 0)),
                  pl.BlockSpec((NC, NB, H), lambda i: (0, i, 0)),
                  full((H, H)), full((H, H)), full((1, H)),
                  full((H, H)), full((1, H))],
        out_specs=pl.BlockSpec((NB, H), lambda i: (i, 0)),
        out_shape=jax.ShapeDtypeStruct((N, H), jnp.float32),
    )(h, mpart, U, V, hb1, hW2, hb2)


def kernel(atomic_numbers, pos, edge_index, edge_attr, emb,
           e_w1, e_b1, e_w2, e_b2,
           h_w1, h_b1, h_w2, h_b2,
           x_w1, x_b1, x_w2, x_b2):
    del edge_attr
    ids2d = atomic_numbers.astype(jnp.int32).reshape(N, 1)
    embp = jnp.zeros((H, H), jnp.float32).at[:emb.shape[0]].set(emb)
    src = edge_index[0].astype(jnp.int32)
    dst = edge_index[1].astype(jnp.int32)
    zh = jnp.zeros((N, H), jnp.float32)

    h = _embed(ids2d, embp)
    x4 = jnp.pad(pos, ((0, 0), (0, 1)))

    for l in range(e_w1.shape[0]):
        A = e_w1[l, :H]
        B = e_w1[l, H:2 * H]
        crow = e_w1[l, 2 * H:2 * H + 1]
        b1 = e_b1[l].reshape(1, H)
        W2 = e_w2[l]
        b2 = e_b2[l].reshape(1, H)
        xw1 = x_w1[l]
        xb1 = x_b1[l].reshape(1, H)
        xw2 = x_w2[l]
        xb2 = x_b2[l].reshape(1, 1)
        U = h_w1[l, :H]
        V = h_w1[l, H:]
        hb1 = h_b1[l].reshape(1, H)
        hW2 = h_w2[l]
        hb2 = h_b2[l].reshape(1, H)

        hi, hj, g = _sc_gather(h, x4.reshape(-1), src, dst)
        mij, parr = _edge_mlp(hi, hj, g, A, B, crow, b1, W2, b2,
                              xw1, xb1, xw2, xb2)
        mpart = _sc_scatter_m(mij, dst, zh)
        xpart = _sc_scatter_x(parr, dst)
        x4 = _xsum(x4, xpart)
        h = _node_update(h, mpart, U, V, hb1, hW2, hb2)

    return (h, x4[:, :3])


# speedup vs baseline: 4.9345x; 1.5022x over previous
import functools

import jax
import jax.numpy as jnp
from jax import lax
from jax.experimental import pallas as pl
from jax.experimental.pallas import tpu as pltpu
from jax.experimental.pallas import tpu_sc as plsc

N = 10000
E = 320000
H = 128
NC = 2
NS = 16
NW = NC * NS
EPT = E // NW
CH = 80
NCH = EPT // CH
NPT = N // NS
NB = 400
NGRID = N // NB
EB = 2560
EGRID = E // EB
PW = 16

def _sc_mesh():
    return plsc.VectorSubcoreMesh(core_axis_name="c", subcore_axis_name="s",
                                  num_cores=NC, num_subcores=NS)


def _emb_body(ids_ref, emb_ref, out_ref):
    ids = ids_ref[...]
    cols = lax.broadcasted_iota(jnp.int32, (NB, H), 1)
    onehot = (cols == ids).astype(jnp.float32)
    out_ref[...] = jnp.dot(onehot, emb_ref[...],
                           preferred_element_type=jnp.float32)


def _embed(ids2d, embp):
    return pl.pallas_call(
        _emb_body,
        grid=(NGRID,),
        in_specs=[pl.BlockSpec((NB, 1), lambda i: (i, 0)),
                  pl.BlockSpec((H, H), lambda i: (0, 0))],
        out_specs=pl.BlockSpec((NB, H), lambda i: (i, 0)),
        out_shape=jax.ShapeDtypeStruct((N, H), jnp.float32),
    )(ids2d, embp)


def _gather_body(h_hbm, xf_hbm, src_hbm, dst_hbm,
                 hi_hbm, hj_hbm, g_hbm,
                 xtab, sidx, didx,
                 hibuf0, hjbuf0, gbuf0, hibuf1, hjbuf1, gbuf1,
                 gsem0, gsem1, wsem0, wsem1):
    c = lax.axis_index("c")
    s = lax.axis_index("s")
    base = (c * NS + s) * EPT
    pltpu.sync_copy(xf_hbm, xtab)
    pltpu.sync_copy(src_hbm.at[pl.ds(base, EPT)], sidx)
    pltpu.sync_copy(dst_hbm.at[pl.ds(base, EPT)], didx)
    iota16 = lax.iota(jnp.int32, 16)
    hibuf = (hibuf0, hibuf1)
    hjbuf = (hjbuf0, hjbuf1)
    gbuf = (gbuf0, gbuf1)
    gsem = (gsem0, gsem1)
    wsem = (wsem0, wsem1)

    def ioff(j):
        return pl.multiple_of(j * CH, 8)

    def hoff(j):
        return pl.multiple_of(base + j * CH, 8)

    def issue_gather(j, b):
        pltpu.async_copy(h_hbm.at[sidx.at[pl.ds(ioff(j), CH)]], hjbuf[b], gsem[b])
        pltpu.async_copy(h_hbm.at[didx.at[pl.ds(ioff(j), CH)]], hibuf[b], gsem[b])

    def wait_gather(j, b):
        pltpu.make_async_copy(h_hbm.at[sidx.at[pl.ds(ioff(j), CH)]],
                              hjbuf[b], gsem[b]).wait()
        pltpu.make_async_copy(h_hbm.at[didx.at[pl.ds(ioff(j), CH)]],
                              hibuf[b], gsem[b]).wait()

    def issue_wb(j, b):
        pltpu.async_copy(hibuf[b], hi_hbm.at[pl.ds(hoff(j), CH)], wsem[b])
        pltpu.async_copy(hjbuf[b], hj_hbm.at[pl.ds(hoff(j), CH)], wsem[b])
        pltpu.async_copy(gbuf[b], g_hbm.at[pl.ds(hoff(j), CH)], wsem[b])

    def wait_wb(j, b):
        pltpu.make_async_copy(hibuf[b], hi_hbm.at[pl.ds(hoff(j), CH)], wsem[b]).wait()
        pltpu.make_async_copy(hjbuf[b], hj_hbm.at[pl.ds(hoff(j), CH)], wsem[b]).wait()
        pltpu.make_async_copy(gbuf[b], g_hbm.at[pl.ds(hoff(j), CH)], wsem[b]).wait()

    def compute_g(j, b):
        for q in range(CH // 16):
            o = pl.multiple_of(j * CH, 8) + q * 16
            s16 = sidx[pl.ds(o, 16)]
            d16 = didx[pl.ds(o, 16)]
            comps = []
            for cc in range(3):
                xs = plsc.load_gather(xtab, [s16 * 4 + cc])
                xd = plsc.load_gather(xtab, [d16 * 4 + cc])
                comps.append(xs - xd)
            r2 = comps[0] * comps[0] + comps[1] * comps[1] + comps[2] * comps[2]
            r16 = q * 16 + iota16
            for cc in range(3):
                plsc.store_scatter(gbuf[b], [r16, iota16 * 0 + cc], comps[cc])
            plsc.store_scatter(gbuf[b], [r16, iota16 * 0 + 3], r2)

    issue_gather(0, 0)

    def dbl(jj, carry):
        j0 = jj * 2
        wait_gather(j0, 0)
        compute_g(j0, 0)

        @pl.when(jj >= 1)
        def _():
            wait_wb(j0 - 1, 1)

        issue_gather(j0 + 1, 1)
        issue_wb(j0, 0)

        j1 = j0 + 1
        wait_gather(j1, 1)
        compute_g(j1, 1)
        wait_wb(j1 - 1, 0)
        issue_gather(j1 + 1, 0)
        issue_wb(j1, 1)
        return carry

    lax.fori_loop(0, (NCH - 1) // 2, dbl, 0)
    jl = NCH - 1
    wait_gather(jl, 0)
    compute_g(jl, 0)
    wait_wb(jl - 1, 1)
    pltpu.sync_copy(hibuf[0], hi_hbm.at[pl.ds(hoff(jl), CH)])
    pltpu.sync_copy(hjbuf[0], hj_hbm.at[pl.ds(hoff(jl), CH)])
    pltpu.sync_copy(gbuf[0], g_hbm.at[pl.ds(hoff(jl), CH)])


def _sc_gather(h, xflat, src, dst):
    f = pl.kernel(
        _gather_body,
        out_type=(jax.ShapeDtypeStruct((E, H), jnp.float32),
                  jax.ShapeDtypeStruct((E, H), jnp.float32),
                  jax.ShapeDtypeStruct((E, 8), jnp.float32)),
        mesh=_sc_mesh(),
        compiler_params=pltpu.CompilerParams(needs_layout_passes=False),
        scratch_types=[pltpu.VMEM((4 * N,), jnp.float32),
                       pltpu.VMEM((EPT,), jnp.int32),
                       pltpu.VMEM((EPT,), jnp.int32),
                       pltpu.VMEM((CH, H), jnp.float32),
                       pltpu.VMEM((CH, H), jnp.float32),
                       pltpu.VMEM((CH, 8), jnp.float32),
                       pltpu.VMEM((CH, H), jnp.float32),
                       pltpu.VMEM((CH, H), jnp.float32),
                       pltpu.VMEM((CH, 8), jnp.float32),
                       pltpu.SemaphoreType.DMA,
                       pltpu.SemaphoreType.DMA,
                       pltpu.SemaphoreType.DMA,
                       pltpu.SemaphoreType.DMA],
    )
    return f(h, xflat, src, dst)


def _edge_body(hi_ref, hj_ref, g_ref, A_ref, B_ref, c_ref, b1_ref,
               W2_ref, b2_ref, xw1_ref, xb1_ref, xw2_ref, xb2_ref,
               mij_ref, p_ref):
    g = g_ref[...]
    r = jnp.sqrt(g[:, 3:4])
    t = (jnp.dot(hi_ref[...], A_ref[...], preferred_element_type=jnp.float32)
         + jnp.dot(hj_ref[...], B_ref[...], preferred_element_type=jnp.float32)
         + r * c_ref[...] + b1_ref[...])
    m = t * jax.nn.sigmoid(t)
    t2 = jnp.dot(m, W2_ref[...], preferred_element_type=jnp.float32) + b2_ref[...]
    m2 = t2 * jax.nn.sigmoid(t2)
    mij_ref[...] = m2
    t3 = jnp.dot(m2, xw1_ref[...], preferred_element_type=jnp.float32) + xb1_ref[...]
    w = t3 * jax.nn.sigmoid(t3)
    Wij = jnp.dot(w, xw2_ref[...], preferred_element_type=jnp.float32) + xb2_ref[...]
    p = Wij * g[:, 0:3]
    p_ref[...] = jnp.concatenate(
        [p, jnp.zeros((EB, PW - 3), jnp.float32)], axis=1)


def _edge_mlp(hi, hj, g, A, B, crow, b1, W2, b2, xw1, xb1, xw2, xb2):
    full = lambda shape: pl.BlockSpec(shape, lambda i: tuple(0 for _ in shape))
    return pl.pallas_call(
        _edge_body,
        grid=(EGRID,),
        in_specs=[pl.BlockSpec((EB, H), lambda i: (i, 0)),
                  pl.BlockSpec((EB, H), lambda i: (i, 0)),
                  pl.BlockSpec((EB, 8), lambda i: (i, 0)),
                  full((H, H)), full((H, H)), full((1, H)), full((1, H)),
                  full((H, H)), full((1, H)),
                  full((H, H)), full((1, H)), full((H, 1)), full((1, 1))],
        out_specs=[pl.BlockSpec((EB, H), lambda i: (i, 0)),
                   pl.BlockSpec((EB, PW), lambda i: (i, 0))],
        out_shape=[jax.ShapeDtypeStruct((E, H), jnp.float32),
                   jax.ShapeDtypeStruct((E, PW), jnp.float32)],
    )(hi, hj, g, A, B, crow, b1, W2, b2, xw1, xb1, xw2, xb2)


def _scatter_m_body(mij_hbm, dst3_hbm, zh_hbm, mpart_hbm,
                    didx2, mbuf0, mbuf1, lsem0, lsem1, macc):
    c = lax.axis_index("c")
    s = lax.axis_index("s")
    wid = c * NS + s
    base = wid * EPT
    rows0 = s * 624

    @pl.when(s < NS - 1)
    def _():
        pltpu.sync_copy(zh_hbm.at[pl.ds(rows0, 624)], macc.at[pl.ds(rows0, 624)])

    @pl.when(s == NS - 1)
    def _():
        pltpu.sync_copy(zh_hbm.at[pl.ds(rows0, 640)], macc.at[pl.ds(rows0, 640)])

    pltpu.sync_copy(dst3_hbm.at[wid], didx2)
    plsc.subcore_barrier()
    mbuf = (mbuf0, mbuf1)
    lsem = (lsem0, lsem1)

    def hoff(j):
        return pl.multiple_of(base + j * CH, 8)

    def issue_load(j, b):
        pltpu.async_copy(mij_hbm.at[pl.ds(hoff(j), CH)], mbuf[b], lsem[b])

    def wait_load(j, b):
        pltpu.make_async_copy(mij_hbm.at[pl.ds(hoff(j), CH)], mbuf[b], lsem[b]).wait()

    issue_load(0, 0)

    def dbl(jj, carry):
        j0 = jj * 2
        issue_load(j0 + 1, 1)
        wait_load(j0, 0)
        pltpu.sync_copy(mbuf[0], macc.at[didx2.at[j0]], add=True)
        j1 = j0 + 1
        issue_load(j1 + 1, 0)
        wait_load(j1, 1)
        pltpu.sync_copy(mbuf[1], macc.at[didx2.at[j1]], add=True)
        return carry

    lax.fori_loop(0, (NCH - 1) // 2, dbl, 0)
    jl = NCH - 1
    wait_load(jl, 0)
    pltpu.sync_copy(mbuf[0], macc.at[didx2.at[jl]], add=True)
    plsc.subcore_barrier()

    @pl.when(s < NS - 1)
    def _():
        pltpu.sync_copy(macc.at[pl.ds(rows0, 624)],
                        mpart_hbm.at[c].at[pl.ds(rows0, 624)])

    @pl.when(s == NS - 1)
    def _():
        pltpu.sync_copy(macc.at[pl.ds(rows0, 640)],
                        mpart_hbm.at[c].at[pl.ds(rows0, 640)])


def _sc_scatter_m(mij, dst3, zh):
    f = pl.kernel(
        _scatter_m_body,
        out_type=jax.ShapeDtypeStruct((NC, N, H), jnp.float32),
        mesh=_sc_mesh(),
        compiler_params=pltpu.CompilerParams(needs_layout_passes=False),
        scratch_types=[pltpu.VMEM((NCH, CH), jnp.int32),
                       pltpu.VMEM((CH, H), jnp.float32),
                       pltpu.VMEM((CH, H), jnp.float32),
                       pltpu.SemaphoreType.DMA,
                       pltpu.SemaphoreType.DMA,
                       pltpu.VMEM_SHARED((N, H), jnp.float32)],
    )
    return f(mij, dst3, zh)


def _scatter_x_body(pf_hbm, dst_hbm, xpart_hbm,
                    didx, pbuf0, pbuf1, psem0, psem1, xacc):
    c = lax.axis_index("c")
    s = lax.axis_index("s")
    wid = c * NS + s
    base = wid * EPT
    iota16 = lax.iota(jnp.int32, 16)
    pltpu.sync_copy(dst_hbm.at[pl.ds(base, EPT)], didx)

    def zloop(i, carry):
        plsc.store_scatter(xacc, [i * 16 + iota16], jnp.zeros((16,), jnp.float32))
        return carry

    lax.fori_loop(0, (N * 4) // 16, zloop, 0)
    pbuf = (pbuf0, pbuf1)
    psem = (psem0, psem1)

    def hoff(j):
        return pl.multiple_of(base + j * CH, 8)

    def issue_load(j, b):
        pltpu.async_copy(pf_hbm.at[pl.ds(hoff(j), CH)], pbuf[b], psem[b])

    def wait_load(j, b):
        pltpu.make_async_copy(pf_hbm.at[pl.ds(hoff(j), CH)], pbuf[b], psem[b]).wait()

    def compute(j, b):
        for q in range(CH // 16):
            o = pl.multiple_of(j * CH, 8) + q * 16
            d16 = didx[pl.ds(o, 16)]
            r16 = q * 16 + iota16
            for cc in range(3):
                val = plsc.load_gather(pbuf[b], [r16, iota16 * 0 + cc])
                plsc.addupdate_scatter(xacc, [d16 * 4 + cc], val)

    issue_load(0, 0)

    def dbl(jj, carry):
        j0 = jj * 2
        issue_load(j0 + 1, 1)
        wait_load(j0, 0)
        compute(j0, 0)
        j1 = j0 + 1
        issue_load(j1 + 1, 0)
        wait_load(j1, 1)
        compute(j1, 1)
        return carry

    lax.fori_loop(0, (NCH - 1) // 2, dbl, 0)
    jl = NCH - 1
    wait_load(jl, 0)
    compute(jl, 0)
    pltpu.sync_copy(xacc, xpart_hbm.at[wid])


def _sc_scatter_x(parr, dst):
    f = pl.kernel(
        _scatter_x_body,
        out_type=jax.ShapeDtypeStruct((NW, N * 4), jnp.float32),
        mesh=_sc_mesh(),
        compiler_params=pltpu.CompilerParams(needs_layout_passes=False),
        scratch_types=[pltpu.VMEM((EPT,), jnp.int32),
                       pltpu.VMEM((CH, PW), jnp.float32),
                       pltpu.VMEM((CH, PW), jnp.float32),
                       pltpu.SemaphoreType.DMA,
                       pltpu.SemaphoreType.DMA,
                       pltpu.VMEM((N * 4,), jnp.float32)],
    )
    return f(parr, dst)


def _xsum_body(x_ref, xp_ref, out_ref):
    out_ref[...] = x_ref[...] + jnp.sum(xp_ref[...], axis=0)


def _xsum(x4, xpart):
    XL = 1600
    x3 = x4.reshape(NGRID, 1, XL)
    xp4 = xpart.reshape(NW, NGRID, 1, XL)
    out = pl.pallas_call(
        _xsum_body,
        grid=(NGRID,),
        in_specs=[pl.BlockSpec((1, 1, XL), lambda i: (i, 0, 0)),
                  pl.BlockSpec((NW, 1, 1, XL), lambda i: (0, i, 0, 0))],
        out_specs=pl.BlockSpec((1, 1, XL), lambda i: (i, 0, 0)),
        out_shape=jax.ShapeDtypeStruct((NGRID, 1, XL), jnp.float32),
    )(x3, xp4)
    return out.reshape(N, 4)


def _node_body(h_ref, mp_ref, U_ref, V_ref, b1_ref,
               W2_ref, b2_ref, hout_ref):
    m_i = mp_ref[0] + mp_ref[1]
    t = (jnp.dot(h_ref[...], U_ref[...], preferred_element_type=jnp.float32)
         + jnp.dot(m_i, V_ref[...], preferred_element_type=jnp.float32)
         + b1_ref[...])
    hh = t * jax.nn.sigmoid(t)
    hout_ref[...] = (h_ref[...]
                     + jnp.dot(hh, W2_ref[...], preferred_element_type=jnp.float32)
                     + b2_ref[...])


def _node_update(h, mpart, U, V, hb1, hW2, hb2):
    full = lambda shape: pl.BlockSpec(shape, lambda i: tuple(0 for _ in shape))
    return pl.pallas_call(
        _node_body,
        grid=(NGRID,),
        in_specs=[pl.BlockSpec((NB, H), lambda i: (i, 0)),
                  pl.BlockSpec((NC, NB, H), lambda i: (0, i, 0)),
                  full((H, H)), full((H, H)), full((1, H)),
                  full((H, H)), full((1, H))],
        out_specs=pl.BlockSpec((NB, H), lambda i: (i, 0)),
        out_shape=jax.ShapeDtypeStruct((N, H), jnp.float32),
    )(h, mpart, U, V, hb1, hW2, hb2)


def kernel(atomic_numbers, pos, edge_index, edge_attr, emb,
           e_w1, e_b1, e_w2, e_b2,
           h_w1, h_b1, h_w2, h_b2,
           x_w1, x_b1, x_w2, x_b2):
    del edge_attr
    ids2d = atomic_numbers.astype(jnp.int32).reshape(N, 1)
    embp = jnp.zeros((H, H), jnp.float32).at[:emb.shape[0]].set(emb)
    src = edge_index[0].astype(jnp.int32)
    dst = edge_index[1].astype(jnp.int32)
    dst3 = dst.reshape(NW, NCH, CH)
    zh = jnp.zeros((N, H), jnp.float32)

    h = _embed(ids2d, embp)
    x4 = jnp.pad(pos, ((0, 0), (0, 1)))

    for l in range(e_w1.shape[0]):
        A = e_w1[l, :H]
        B = e_w1[l, H:2 * H]
        crow = e_w1[l, 2 * H:2 * H + 1]
        b1 = e_b1[l].reshape(1, H)
        W2 = e_w2[l]
        b2 = e_b2[l].reshape(1, H)
        xw1 = x_w1[l]
        xb1 = x_b1[l].reshape(1, H)
        xw2 = x_w2[l]
        xb2 = x_b2[l].reshape(1, 1)
        U = h_w1[l, :H]
        V = h_w1[l, H:]
        hb1 = h_b1[l].reshape(1, H)
        hW2 = h_w2[l]
        hb2 = h_b2[l].reshape(1, H)

        hi, hj, g = _sc_gather(h, x4.reshape(-1), src, dst)
        mij, parr = _edge_mlp(hi, hj, g, A, B, crow, b1, W2, b2,
                              xw1, xb1, xw2, xb2)
        mpart = _sc_scatter_m(mij, dst3, zh)
        xpart = _sc_scatter_x(parr, dst)
        x4 = _xsum(x4, xpart)
        h = _node_update(h, mpart, U, V, hb1, hW2, hb2)

    return (h, x4[:, :3])
